# baseline (speedup 1.0000x reference)
import functools

import jax
import jax.numpy as jnp
from jax.experimental import pallas as pl

B, SEQ, D = 4, 4096, 160
MARK_AX = 0
OP_ADD = 1
OP_SUB = 2
ALU_LO = 16
ALU_HI = 32
AX_CARRY_LO = 48
AX_CARRY_HI = 64
OUTPUT_LO = 112
OUTPUT_HI = 128

ROWS = 2048


def _alu_body(x_ref, o_ref):
    x = x_ref[...]
    n = x.shape[0]
    lane = jax.lax.broadcasted_iota(jnp.int32, (n, 16), 1)

    def decode(base):
        f = x[:, base:base + 16] > 0.5
        idx = jnp.min(jnp.where(f, lane, 16), axis=1, keepdims=True)
        return jnp.where(idx == 16, 0, idx)

    a_lo = decode(ALU_LO)
    a_hi = decode(ALU_HI)
    b_lo = decode(AX_CARRY_LO)
    b_hi = decode(AX_CARRY_HI)

    mark = x[:, MARK_AX:MARK_AX + 1] > 0.5
    is_add = x[:, OP_ADD:OP_ADD + 1] > 0.5
    is_sub = jnp.logical_and(jnp.logical_not(is_add), x[:, OP_SUB:OP_SUB + 1] > 0.5)
    active = jnp.logical_and(mark, jnp.logical_or(is_add, is_sub))

    sum_lo = a_lo + b_lo
    add_r_lo = jnp.mod(sum_lo, 16)
    carry = sum_lo // 16
    add_r_hi = jnp.mod(a_hi + b_hi + carry, 16)

    diff_lo = a_lo - b_lo
    sub_r_lo = jnp.mod(diff_lo, 16)
    borrow = (diff_lo < 0).astype(jnp.int32)
    sub_r_hi = jnp.mod(a_hi - b_hi - borrow, 16)

    r_lo = jnp.where(is_add, add_r_lo, sub_r_lo)
    r_hi = jnp.where(is_add, add_r_hi, sub_r_hi)

    amp = jnp.where(active, 2.0, 0.0).astype(x.dtype)
    oh_lo = jnp.where(lane == r_lo, amp, 0.0)
    oh_hi = jnp.where(lane == r_hi, amp, 0.0)

    o_ref[...] = x
    o_ref[:, OUTPUT_LO:OUTPUT_LO + 16] = x[:, OUTPUT_LO:OUTPUT_LO + 16] + oh_lo
    o_ref[:, OUTPUT_HI:OUTPUT_HI + 16] = x[:, OUTPUT_HI:OUTPUT_HI + 16] + oh_hi


@jax.jit
def kernel(x_bd):
    x = x_bd.reshape(B * SEQ, D)
    out = pl.pallas_call(
        _alu_body,
        grid=(B * SEQ // ROWS,),
        in_specs=[pl.BlockSpec((ROWS, D), lambda i: (i, 0))],
        out_specs=pl.BlockSpec((ROWS, D), lambda i: (i, 0)),
        out_shape=jax.ShapeDtypeStruct((B * SEQ, D), x.dtype),
    )(x)
    return out.reshape(B, SEQ, D)
